# Initial kernel scaffold; baseline (speedup 1.0000x reference)
import jax
import jax.numpy as jnp
from jax.experimental import pallas as pl


def kernel(x, x_shuffle, edge_index, W1, b1, W2, b2, Wb, bb):
    raise NotImplementedError("write your pallas kernel here")


# speedup vs baseline: 6.2587x; 6.2587x over previous
import jax
import jax.numpy as jnp
from jax import lax
from jax.experimental import pallas as pl
from jax.experimental.pallas import tpu as pltpu
from jax.experimental.pallas import tpu_sc as plsc

_NC = 2
_NS = 16
_B = 128
_RB = 1024


def _mesh():
    return plsc.VectorSubcoreMesh(
        core_axis_name="c", subcore_axis_name="s",
        num_cores=_NC, num_subcores=_NS)


def _sc_degree(n_p, nb):
    rpt = n_p // _NS
    ncp = rpt // _B

    def body(dst_f, ones_hbm, out, dacc, idst, ones_v, stage, sem):
        cid = lax.axis_index("c")
        sid = lax.axis_index("s")
        w = cid * _NS + sid
        r0 = pl.multiple_of(sid * rpt, 8)
        pltpu.sync_copy(ones_hbm.at[pl.ds(0, _B)], ones_v)
        for t in range(ncp):
            pltpu.sync_copy(ones_v,
                            dacc.at[pl.ds(r0 + t * _B, _B)])
        plsc.subcore_barrier()

        def batch(b, carry):
            off = pl.multiple_of((w * nb + b) * _B, 8)
            pltpu.sync_copy(dst_f.at[pl.ds(off, _B)], idst)
            pltpu.sync_copy(ones_v, dacc.at[idst], add=True)
            return carry
        lax.fori_loop(0, nb, batch, 0)
        plsc.subcore_barrier()
        for t in range(ncp):
            pltpu.sync_copy(dacc.at[pl.ds(r0 + t * _B, _B)], stage)
            pltpu.sync_copy(stage,
                            out.at[cid, pl.ds(r0 + t * _B, _B)])

    return pl.kernel(
        body,
        out_type=jax.ShapeDtypeStruct((_NC, n_p, 16), jnp.float32),
        mesh=_mesh(),
        scratch_types=[
            pltpu.VMEM_SHARED((n_p, 16), jnp.float32),
            pltpu.VMEM((_B,), jnp.int32),
            pltpu.VMEM((_B, 16), jnp.float32),
            pltpu.VMEM((_B, 16), jnp.float32),
            pltpu.SemaphoreType.DMA,
        ],
    )


def _sc_agg(n_p, nb, nchunks):
    rpt = n_p // _NS
    ncp = rpt // _B

    def body(*refs):
        zs = refs[:nchunks]
        src_f = refs[nchunks]
        dst_f = refs[nchunks + 1]
        out = refs[nchunks + 2]
        acc, isrc, idst, rows, sem = refs[nchunks + 3:]
        cid = lax.axis_index("c")
        sid = lax.axis_index("s")
        w = cid * _NS + sid
        r0 = pl.multiple_of(sid * rpt, 8)
        for k in range(nchunks):
            zk = zs[k]
            for t in range(ncp):
                pltpu.sync_copy(zk.at[pl.ds(r0 + t * _B, _B)], rows)
                pltpu.sync_copy(rows, acc.at[pl.ds(r0 + t * _B, _B)])
            plsc.subcore_barrier()

            def batch(b, carry):
                off = pl.multiple_of((w * nb + b) * _B, 8)
                pltpu.sync_copy(src_f.at[pl.ds(off, _B)], isrc)
                pltpu.sync_copy(dst_f.at[pl.ds(off, _B)], idst)
                pltpu.async_copy(zk.at[isrc], rows, sem).wait()
                pltpu.sync_copy(rows, acc.at[idst], add=True)
                return carry
            lax.fori_loop(0, nb, batch, 0)
            plsc.subcore_barrier()
            for t in range(ncp):
                pltpu.sync_copy(acc.at[pl.ds(r0 + t * _B, _B)], rows)
                pltpu.sync_copy(rows,
                                out.at[cid, k, pl.ds(r0 + t * _B, _B)])
            if k + 1 < nchunks:
                plsc.subcore_barrier()

    return pl.kernel(
        body,
        out_type=jax.ShapeDtypeStruct((_NC, nchunks, n_p, 128), jnp.float32),
        mesh=_mesh(),
        scratch_types=[
            pltpu.VMEM_SHARED((n_p, 128), jnp.float32),
            pltpu.VMEM((_B,), jnp.int32),
            pltpu.VMEM((_B,), jnp.int32),
            pltpu.VMEM((_B, 128), jnp.float32),
            pltpu.SemaphoreType.DMA,
        ],
    )


def _tc_prep(x, xs, degp, W1):
    n_p = x.shape[0]
    grid = (n_p // _RB,)

    def body(x_ref, xs_ref, dp_ref, W1_ref, z_ref, dis_ref):
        deg = dp_ref[0] + dp_ref[1] - 1.0
        dis = lax.rsqrt(deg)
        dis_ref[...] = dis
        d1 = dis[:, 0:1]
        zx = jnp.dot(x_ref[...] * d1, W1_ref[...],
                     preferred_element_type=jnp.float32)
        zs = jnp.dot(xs_ref[...] * d1, W1_ref[...],
                     preferred_element_type=jnp.float32)
        z_ref[0] = zx[:, :128]
        z_ref[1] = zx[:, 128:]
        z_ref[2] = zs[:, :128]
        z_ref[3] = zs[:, 128:]

    return pl.pallas_call(
        body,
        grid=grid,
        in_specs=[
            pl.BlockSpec((_RB, 128), lambda i: (i, 0)),
            pl.BlockSpec((_RB, 128), lambda i: (i, 0)),
            pl.BlockSpec((2, _RB, 16), lambda i: (0, i, 0)),
            pl.BlockSpec((128, 256), lambda i: (0, 0)),
        ],
        out_specs=[
            pl.BlockSpec((4, _RB, 128), lambda i: (0, i, 0)),
            pl.BlockSpec((_RB, 16), lambda i: (i, 0)),
        ],
        out_shape=[
            jax.ShapeDtypeStruct((4, n_p, 128), jnp.float32),
            jax.ShapeDtypeStruct((n_p, 16), jnp.float32),
        ],
    )(x, xs, degp, W1)


def _tc_mid(p1, z1, dis16, b1_2d, W2):
    n_p = z1.shape[1]
    grid = (n_p // _RB,)

    def body(p_ref, z_ref, dis_ref, b1_ref, W2_ref, z2_ref):
        dis = dis_ref[...][:, 0:1]
        b1r = b1_ref[0:1, :]
        accx = jnp.concatenate(
            [p_ref[0, 0] + p_ref[1, 0] - z_ref[0],
             p_ref[0, 1] + p_ref[1, 1] - z_ref[1]], axis=1)
        accs = jnp.concatenate(
            [p_ref[0, 2] + p_ref[1, 2] - z_ref[2],
             p_ref[0, 3] + p_ref[1, 3] - z_ref[3]], axis=1)
        hx = dis * jnp.maximum(dis * accx + b1r, 0.0)
        hs = dis * jnp.maximum(dis * accs + b1r, 0.0)
        z2_ref[0] = jnp.dot(hx, W2_ref[...],
                            preferred_element_type=jnp.float32)
        z2_ref[1] = jnp.dot(hs, W2_ref[...],
                            preferred_element_type=jnp.float32)

    return pl.pallas_call(
        body,
        grid=grid,
        in_specs=[
            pl.BlockSpec((2, 4, _RB, 128), lambda i: (0, 0, i, 0)),
            pl.BlockSpec((4, _RB, 128), lambda i: (0, i, 0)),
            pl.BlockSpec((_RB, 16), lambda i: (i, 0)),
            pl.BlockSpec((1, 256), lambda i: (0, 0)),
            pl.BlockSpec((256, 128), lambda i: (0, 0)),
        ],
        out_specs=pl.BlockSpec((2, _RB, 128), lambda i: (0, i, 0)),
        out_shape=jax.ShapeDtypeStruct((2, n_p, 128), jnp.float32),
    )(p1, z1, dis16, b1_2d, W2)


def _tc_final(p2, z2, dis16, b2_2d, Wb, bb_2d):
    n_p = z2.shape[1]
    grid = (n_p // _RB,)

    def body(p_ref, z_ref, dis_ref, b2_ref, Wb_ref, bb_ref, out_ref):
        dis = dis_ref[...][:, 0:1]
        b2r = b2_ref[0:1, :]
        h1 = jnp.maximum(
            dis * (p_ref[0, 0] + p_ref[1, 0] - z_ref[0]) + b2r, 0.0)
        h2 = jnp.maximum(
            dis * (p_ref[0, 1] + p_ref[1, 1] - z_ref[1]) + b2r, 0.0)
        c = jax.nn.sigmoid(jnp.mean(h1, axis=1, keepdims=True))
        wcol = jnp.sum(Wb_ref[0], axis=1, keepdims=True)
        bb0 = bb_ref[0, 0]
        s1 = c * jnp.dot(h1, wcol, preferred_element_type=jnp.float32) + bb0
        s2 = c * jnp.dot(h2, wcol, preferred_element_type=jnp.float32) + bb0
        out_ref[...] = jnp.concatenate([s1, s2], axis=1)

    return pl.pallas_call(
        body,
        grid=grid,
        in_specs=[
            pl.BlockSpec((2, 2, _RB, 128), lambda i: (0, 0, i, 0)),
            pl.BlockSpec((2, _RB, 128), lambda i: (0, i, 0)),
            pl.BlockSpec((_RB, 16), lambda i: (i, 0)),
            pl.BlockSpec((1, 128), lambda i: (0, 0)),
            pl.BlockSpec((1, 128, 128), lambda i: (0, 0, 0)),
            pl.BlockSpec((1, 128), lambda i: (0, 0)),
        ],
        out_specs=pl.BlockSpec((_RB, 2), lambda i: (i, 0)),
        out_shape=jax.ShapeDtypeStruct((n_p, 2), jnp.float32),
    )(p2, z2, dis16, b2_2d, Wb, bb_2d)


def kernel(x, x_shuffle, edge_index, W1, b1, W2, b2, Wb, bb):
    n = x.shape[0]
    e = edge_index.shape[1]
    n_p = -(-n // (_NS * _B)) * (_NS * _B)
    per_tile = -(-e // (_NC * _NS * _B)) * _B
    nb = per_tile // _B
    e_pad = _NC * _NS * per_tile
    src = edge_index[0].astype(jnp.int32)
    dst = edge_index[1].astype(jnp.int32)
    src_f = jnp.concatenate([src, jnp.zeros((e_pad - e,), jnp.int32)])
    dst_f = jnp.concatenate([dst, jnp.full((e_pad - e,), n, jnp.int32)])
    xp = jnp.pad(x, ((0, n_p - n), (0, 0)))
    xsp = jnp.pad(x_shuffle, ((0, n_p - n), (0, 0)))
    ones16 = jnp.ones((n_p, 16), jnp.float32)

    degp = _sc_degree(n_p, nb)(dst_f, ones16)
    z1, dis16 = _tc_prep(xp, xsp, degp, W1)
    z1c = [z1[k] for k in range(4)]
    p1 = _sc_agg(n_p, nb, 4)(*z1c, src_f, dst_f)
    z2 = _tc_mid(p1, z1, dis16, b1.reshape(1, 256), W2)
    z2c = [z2[k] for k in range(2)]
    p2 = _sc_agg(n_p, nb, 2)(*z2c, src_f, dst_f)
    bb_2d = jnp.broadcast_to(bb.reshape(1, 1), (1, 128))
    out2 = _tc_final(p2, z2, dis16, b2.reshape(1, 128), Wb, bb_2d)
    return jnp.concatenate([out2[:n, 0], out2[:n, 1]])
